# Initial kernel scaffold; baseline (speedup 1.0000x reference)
import functools

import jax
import jax.numpy as jnp
from jax import lax
from jax.experimental import pallas as pl
from jax.experimental.pallas import tpu as pltpu
from jax.experimental.pallas import tpu_sc as plsc

N = 10000
E = 320000
D = 128

NC = 2
NS = 16
NW = NC * NS
EW = E // NW
K = 80
NCHUNK = EW // K
ROWS_PER_TILE = N // NS
ZROWS = 125


def _sc_segment_sum(h, src, dst):
    mesh = plsc.VectorSubcoreMesh(core_axis_name="c", subcore_axis_name="s")

    @functools.partial(
        pl.kernel,
        out_type=(
            jax.ShapeDtypeStruct((NC, N, D), jnp.float32),
            jax.ShapeDtypeStruct((NC, N, 16), jnp.float32),
        ),
        mesh=mesh,
        scratch_types=[
            pltpu.VMEM((K,), jnp.int32),
            pltpu.VMEM((K,), jnp.int32),
            pltpu.VMEM((K, D), jnp.float32),
            pltpu.VMEM((K, 16), jnp.float32),
            pltpu.VMEM((ZROWS, D), jnp.float32),
            pltpu.VMEM((ROWS_PER_TILE, 16), jnp.float32),
            pltpu.VMEM_SHARED((N, D), jnp.float32),
            pltpu.VMEM_SHARED((N, 16), jnp.float32),
            pltpu.SemaphoreType.DMA,
        ],
    )
    def k(h_hbm, src_hbm, dst_hbm, agg_out, deg_out,
          si, di, rows, ones_v, zb, zb16, agg_sh, deg_sh, sem):
        c = lax.axis_index("c")
        s = lax.axis_index("s")
        w = c * NS + s

        zeros16 = jnp.zeros((16,), jnp.float32)
        ones16 = jnp.ones((16,), jnp.float32)

        def zb_row(i, _):
            def zb_col(j, _):
                zb[i, pl.ds(pl.multiple_of(j * 16, 16), 16)] = zeros16
                return 0
            return lax.fori_loop(0, D // 16, zb_col, 0)

        lax.fori_loop(0, ZROWS, zb_row, 0)

        def zb16_row(i, _):
            zb16[i, :] = zeros16
            return 0

        lax.fori_loop(0, ROWS_PER_TILE, zb16_row, 0)

        def ones_row(i, _):
            ones_v[i, :] = ones16
            return 0

        lax.fori_loop(0, K, ones_row, 0)

        for t in range(ROWS_PER_TILE // ZROWS):
            pltpu.sync_copy(zb, agg_sh.at[pl.ds(s * ROWS_PER_TILE + t * ZROWS, ZROWS)])
        pltpu.sync_copy(zb16, deg_sh.at[pl.ds(s * ROWS_PER_TILE, ROWS_PER_TILE)])
        plsc.subcore_barrier()

        base = w * EW

        def chunk(j, _):
            off = pl.multiple_of(base + j * K, 8)
            pltpu.sync_copy(src_hbm.at[pl.ds(off, K)], si)
            pltpu.sync_copy(dst_hbm.at[pl.ds(off, K)], di)
            pltpu.async_copy(h_hbm.at[si], rows, sem).wait()
            pltpu.sync_copy(rows, agg_sh.at[di], add=True)
            pltpu.sync_copy(ones_v, deg_sh.at[di], add=True)
            return 0

        lax.fori_loop(0, NCHUNK, chunk, 0)
        plsc.subcore_barrier()

        r0 = s * ROWS_PER_TILE
        pltpu.sync_copy(agg_sh.at[pl.ds(r0, ROWS_PER_TILE)],
                        agg_out.at[c, pl.ds(r0, ROWS_PER_TILE)])
        pltpu.sync_copy(deg_sh.at[pl.ds(r0, ROWS_PER_TILE)],
                        deg_out.at[c, pl.ds(r0, ROWS_PER_TILE)])

    return k(h, src, dst)


def _tc_body(h_ref, agg_ref, deg_ref, w1_ref, w2_ref, b_ref, g_ref, be_ref,
             out_ref):
    h = h_ref[...]
    agg = agg_ref[0] + agg_ref[1]
    deg = deg_ref[0] + deg_ref[1]
    degc = jnp.maximum(deg[:, 0:1], 1.0)
    cfeat = agg / degc
    z = (jnp.dot(h, w1_ref[...], preferred_element_type=jnp.float32)
         + jnp.dot(cfeat, w2_ref[...], preferred_element_type=jnp.float32)
         + b_ref[...])
    n2 = jnp.sum(z * z, axis=1, keepdims=True)
    z = z * lax.rsqrt(jnp.maximum(n2, 1e-24))
    r = jnp.maximum(z, 0.0)
    mean = jnp.mean(r, axis=0, keepdims=True)
    var = jnp.mean((r - mean) * (r - mean), axis=0, keepdims=True)
    out_ref[...] = h + (r - mean) * lax.rsqrt(var + 1e-5) * g_ref[...] + be_ref[...]


def kernel(h, edge_index, W, b, gamma, beta):
    src = edge_index[0]
    dst = edge_index[1]
    agg_part, deg_part = _sc_segment_sum(h, src, dst)
    w1 = W[:, :D].T
    w2 = W[:, D:].T
    out = pl.pallas_call(
        _tc_body,
        out_shape=jax.ShapeDtypeStruct((N, D), jnp.float32),
    )(h, agg_part, deg_part, w1, w2,
      b.reshape(1, D), gamma.reshape(1, D), beta.reshape(1, D))
    return out


# speedup vs baseline: 3.4561x; 3.4561x over previous
import functools

import jax
import jax.numpy as jnp
from jax import lax
from jax.experimental import pallas as pl
from jax.experimental.pallas import tpu as pltpu
from jax.experimental.pallas import tpu_sc as plsc

N = 10000
E = 320000
D = 128

NS = 16
EW = E // NS
K = 80
NCHUNK = EW // K
WB_TILES = 10
WB_ROWS = N // WB_TILES
ZROWS = 8
BLK = 624
BLK_LAST = N - 15 * BLK


def _sc_segment_mean(h, src, dst):
    mesh = plsc.VectorSubcoreMesh(core_axis_name="c", subcore_axis_name="s",
                                  num_cores=1)

    @functools.partial(
        pl.kernel,
        out_type=jax.ShapeDtypeStruct((N, D), jnp.float32),
        mesh=mesh,
        compiler_params=pltpu.CompilerParams(needs_layout_passes=False),
        scratch_types=[
            pltpu.VMEM((K,), jnp.int32),
            pltpu.VMEM((K,), jnp.int32),
            pltpu.VMEM((K, D), jnp.float32),
            pltpu.VMEM((ZROWS, D), jnp.float32),
            pltpu.VMEM((N,), jnp.float32),
            pltpu.VMEM((BLK_LAST,), jnp.float32),
            pltpu.VMEM((BLK_LAST,), jnp.float32),
            pltpu.VMEM((BLK_LAST,), jnp.float32),
            pltpu.VMEM((16, D), jnp.float32),
            pltpu.VMEM_SHARED((N, D), jnp.float32),
            pltpu.VMEM_SHARED((NS * N,), jnp.float32),
            pltpu.SemaphoreType.DMA,
        ],
    )
    def k(h_hbm, src_hbm, dst_hbm, out_hbm,
          si, di, rows, zb, dl, dsum, dtmp, dinv, cbuf, agg_sh, deg_st, sem):
        s = lax.axis_index("s")
        zeros16 = jnp.zeros((16,), jnp.float32)
        ones16 = jnp.ones((16,), jnp.float32)

        for i in range(ZROWS):
            for j in range(D // 16):
                zb[i, pl.ds(j * 16, 16)] = zeros16

        def dl_zero(i, _):
            dl[pl.ds(pl.multiple_of(i * 16, 16), 16)] = zeros16
            return 0
        lax.fori_loop(0, N // 16, dl_zero, 0)

        @pl.when(s < WB_TILES)
        def _():
            def zcopy(t, _):
                r = pl.multiple_of(s * WB_ROWS + t * ZROWS, 8)
                pltpu.sync_copy(zb, agg_sh.at[pl.ds(r, ZROWS)])
                return 0
            lax.fori_loop(0, WB_ROWS // ZROWS, zcopy, 0)

        plsc.subcore_barrier()

        base = s * EW

        def chunk(j, _):
            off = pl.multiple_of(base + j * K, 8)
            pltpu.sync_copy(src_hbm.at[pl.ds(off, K)], si)
            pltpu.sync_copy(dst_hbm.at[pl.ds(off, K)], di)
            pltpu.async_copy(h_hbm.at[si], rows, sem).wait()
            pltpu.sync_copy(rows, agg_sh.at[di], add=True)
            for i in range(K // 16):
                plsc.addupdate_scatter(dl, [di[pl.ds(i * 16, 16)]], ones16)
            return 0

        lax.fori_loop(0, NCHUNK, chunk, 0)

        pltpu.sync_copy(dl, deg_st.at[pl.ds(s * N, N)])
        plsc.subcore_barrier()

        def epilogue(r0, blk):
            pltpu.sync_copy(deg_st.at[pl.ds(r0, blk)], dsum.at[pl.ds(0, blk)])

            def acc_t(t, _):
                o = pl.multiple_of(t * N + r0, 8)
                pltpu.sync_copy(deg_st.at[pl.ds(o, blk)], dtmp.at[pl.ds(0, blk)])
                for i in range(blk // 16):
                    dsum[pl.ds(i * 16, 16)] = (dsum[pl.ds(i * 16, 16)]
                                               + dtmp[pl.ds(i * 16, 16)])
                return 0
            lax.fori_loop(1, NS, acc_t, 0)

            for i in range(blk // 16):
                dinv[pl.ds(i * 16, 16)] = 1.0 / jnp.maximum(
                    dsum[pl.ds(i * 16, 16)], 1.0)

            def rowblk(g, _):
                rb = pl.multiple_of(r0 + g * 16, 8)
                pltpu.sync_copy(agg_sh.at[pl.ds(rb, 16)], cbuf)
                for l in range(16):
                    bv = plsc.load_gather(
                        dinv, [jnp.full((16,), g * 16 + l, jnp.int32)])
                    for j in range(D // 16):
                        cbuf[l, pl.ds(j * 16, 16)] = (
                            cbuf[l, pl.ds(j * 16, 16)] * bv)
                pltpu.sync_copy(cbuf, out_hbm.at[pl.ds(rb, 16)])
                return 0
            lax.fori_loop(0, blk // 16, rowblk, 0)

        @pl.when(s < 15)
        def _():
            epilogue(pl.multiple_of(s * BLK, 8), BLK)

        @pl.when(s == 15)
        def _():
            epilogue(15 * BLK, BLK_LAST)

    return k(h, src, dst)


def _tc_body(h_ref, c_ref, w1_ref, w2_ref, b_ref, g_ref, be_ref, out_ref):
    h = h_ref[...]
    z = (jnp.dot(h, w1_ref[...], preferred_element_type=jnp.float32)
         + jnp.dot(c_ref[...], w2_ref[...], preferred_element_type=jnp.float32)
         + b_ref[...])
    n2 = jnp.sum(z * z, axis=1, keepdims=True)
    z = z * lax.rsqrt(jnp.maximum(n2, 1e-24))
    r = jnp.maximum(z, 0.0)
    mean = jnp.mean(r, axis=0, keepdims=True)
    var = jnp.mean((r - mean) * (r - mean), axis=0, keepdims=True)
    out_ref[...] = (h + (r - mean) * lax.rsqrt(var + 1e-5) * g_ref[...]
                    + be_ref[...])


def kernel(h, edge_index, W, b, gamma, beta):
    src = edge_index[0]
    dst = edge_index[1]
    cfeat = _sc_segment_mean(h, src, dst)
    w1 = W[:, :D].T
    w2 = W[:, D:].T
    out = pl.pallas_call(
        _tc_body,
        out_shape=jax.ShapeDtypeStruct((N, D), jnp.float32),
    )(h, cfeat, w1, w2,
      b.reshape(1, D), gamma.reshape(1, D), beta.reshape(1, D))
    return out


# speedup vs baseline: 5.5986x; 1.6199x over previous
import functools

import jax
import jax.numpy as jnp
from jax import lax
from jax.experimental import pallas as pl
from jax.experimental.pallas import tpu as pltpu
from jax.experimental.pallas import tpu_sc as plsc

N = 10000
E = 320000
D = 128

NS = 16
EW = E // NS
K = 80
NCHUNK = EW // K
WB_TILES = 10
WB_ROWS = N // WB_TILES
ZROWS = 8
BLK = 624
BLK_LAST = N - 15 * BLK


def _sc_segment_mean(h, src, dst):
    mesh = plsc.VectorSubcoreMesh(core_axis_name="c", subcore_axis_name="s",
                                  num_cores=1)

    @functools.partial(
        pl.kernel,
        out_type=jax.ShapeDtypeStruct((N, D), jnp.float32),
        mesh=mesh,
        compiler_params=pltpu.CompilerParams(needs_layout_passes=False),
        scratch_types=[
            pltpu.VMEM((K,), jnp.int32),
            pltpu.VMEM((K,), jnp.int32),
            pltpu.VMEM((K, D), jnp.float32),
            pltpu.VMEM((K,), jnp.int32),
            pltpu.VMEM((K,), jnp.int32),
            pltpu.VMEM((K, D), jnp.float32),
            pltpu.VMEM((ZROWS, D), jnp.float32),
            pltpu.VMEM((N,), jnp.float32),
            pltpu.VMEM((BLK_LAST,), jnp.float32),
            pltpu.VMEM((BLK_LAST,), jnp.float32),
            pltpu.VMEM((BLK_LAST,), jnp.float32),
            pltpu.VMEM((16, D), jnp.float32),
            pltpu.VMEM_SHARED((N, D), jnp.float32),
            pltpu.VMEM_SHARED((NS * N,), jnp.float32),
            pltpu.SemaphoreType.DMA,
            pltpu.SemaphoreType.DMA,
        ],
    )
    def k(h_hbm, src_hbm, dst_hbm, out_hbm,
          si0, di0, rows0, si1, di1, rows1, zb, dl, dsum, dtmp, dinv, cbuf,
          agg_sh, deg_st, sem0, sem1):
        si = (si0, si1)
        di = (di0, di1)
        rows = (rows0, rows1)
        sem = (sem0, sem1)
        s = lax.axis_index("s")
        zeros16 = jnp.zeros((16,), jnp.float32)
        ones16 = jnp.ones((16,), jnp.float32)

        for i in range(ZROWS):
            for j in range(D // 16):
                zb[i, pl.ds(j * 16, 16)] = zeros16

        def dl_zero(i, _):
            dl[pl.ds(pl.multiple_of(i * 16, 16), 16)] = zeros16
            return 0
        lax.fori_loop(0, N // 16, dl_zero, 0)

        @pl.when(s < WB_TILES)
        def _():
            def zcopy(t, _):
                r = pl.multiple_of(s * WB_ROWS + t * ZROWS, 8)
                pltpu.sync_copy(zb, agg_sh.at[pl.ds(r, ZROWS)])
                return 0
            lax.fori_loop(0, WB_ROWS // ZROWS, zcopy, 0)

        plsc.subcore_barrier()

        base = s * EW

        for b in range(2):
            offp = pl.multiple_of(base + b * K, 8)
            pltpu.sync_copy(src_hbm.at[pl.ds(offp, K)], si[b])
            pltpu.sync_copy(dst_hbm.at[pl.ds(offp, K)], di[b])
            pltpu.async_copy(h_hbm.at[si[b]], rows[b], sem[b])

        def chunk(g, _):
            for b in range(2):
                j = 2 * g + b
                pltpu.make_async_copy(h_hbm.at[si[b]], rows[b], sem[b]).wait()
                pltpu.sync_copy(rows[b], agg_sh.at[di[b]], add=True)
                for i in range(K // 16):
                    plsc.addupdate_scatter(dl, [di[b][pl.ds(i * 16, 16)]],
                                           ones16)
                jn = j + 2

                @pl.when(jn < NCHUNK)
                def _():
                    off = pl.multiple_of(base + jn * K, 8)
                    pltpu.sync_copy(src_hbm.at[pl.ds(off, K)], si[b])
                    pltpu.sync_copy(dst_hbm.at[pl.ds(off, K)], di[b])
                    pltpu.async_copy(h_hbm.at[si[b]], rows[b], sem[b])
            return 0

        lax.fori_loop(0, NCHUNK // 2, chunk, 0)

        pltpu.sync_copy(dl, deg_st.at[pl.ds(s * N, N)])
        plsc.subcore_barrier()

        def epilogue(r0, blk):
            pltpu.sync_copy(deg_st.at[pl.ds(r0, blk)], dsum.at[pl.ds(0, blk)])

            def acc_t(t, _):
                o = pl.multiple_of(t * N + r0, 8)
                pltpu.sync_copy(deg_st.at[pl.ds(o, blk)], dtmp.at[pl.ds(0, blk)])
                for i in range(blk // 16):
                    dsum[pl.ds(i * 16, 16)] = (dsum[pl.ds(i * 16, 16)]
                                               + dtmp[pl.ds(i * 16, 16)])
                return 0
            lax.fori_loop(1, NS, acc_t, 0)

            for i in range(blk // 16):
                dinv[pl.ds(i * 16, 16)] = 1.0 / jnp.maximum(
                    dsum[pl.ds(i * 16, 16)], 1.0)

            def rowblk(g, _):
                rb = pl.multiple_of(r0 + g * 16, 8)
                pltpu.sync_copy(agg_sh.at[pl.ds(rb, 16)], cbuf)
                for l in range(16):
                    bv = plsc.load_gather(
                        dinv, [jnp.full((16,), g * 16 + l, jnp.int32)])
                    for j in range(D // 16):
                        cbuf[l, pl.ds(j * 16, 16)] = (
                            cbuf[l, pl.ds(j * 16, 16)] * bv)
                pltpu.sync_copy(cbuf, out_hbm.at[pl.ds(rb, 16)])
                return 0
            lax.fori_loop(0, blk // 16, rowblk, 0)

        @pl.when(s < 15)
        def _():
            epilogue(pl.multiple_of(s * BLK, 8), BLK)

        @pl.when(s == 15)
        def _():
            epilogue(15 * BLK, BLK_LAST)

    return k(h, src, dst)


def _tc_body(h_ref, c_ref, w1_ref, w2_ref, b_ref, g_ref, be_ref, out_ref):
    h = h_ref[...]
    z = (jnp.dot(h, w1_ref[...], preferred_element_type=jnp.float32)
         + jnp.dot(c_ref[...], w2_ref[...], preferred_element_type=jnp.float32)
         + b_ref[...])
    n2 = jnp.sum(z * z, axis=1, keepdims=True)
    z = z * lax.rsqrt(jnp.maximum(n2, 1e-24))
    r = jnp.maximum(z, 0.0)
    mean = jnp.mean(r, axis=0, keepdims=True)
    var = jnp.mean((r - mean) * (r - mean), axis=0, keepdims=True)
    out_ref[...] = (h + (r - mean) * lax.rsqrt(var + 1e-5) * g_ref[...]
                    + be_ref[...])


def kernel(h, edge_index, W, b, gamma, beta):
    src = edge_index[0]
    dst = edge_index[1]
    cfeat = _sc_segment_mean(h, src, dst)
    w1 = W[:, :D].T
    w2 = W[:, D:].T
    out = pl.pallas_call(
        _tc_body,
        out_shape=jax.ShapeDtypeStruct((N, D), jnp.float32),
    )(h, cfeat, w1, w2,
      b.reshape(1, D), gamma.reshape(1, D), beta.reshape(1, D))
    return out


# speedup vs baseline: 7.7054x; 1.3763x over previous
import functools

import jax
import jax.numpy as jnp
from jax import lax
from jax.experimental import pallas as pl
from jax.experimental.pallas import tpu as pltpu
from jax.experimental.pallas import tpu_sc as plsc

N = 10000
E = 320000
D = 128

NS = 16
EW = E // NS
K = 80
NCHUNK = EW // K
SCH = 25
NSUP = NCHUNK // SCH
WB_TILES = 10
WB_ROWS = N // WB_TILES
ZROWS = 8
BLK = 624
BLK_LAST = N - 15 * BLK


def _sc_segment_mean(h, src, dst):
    mesh = plsc.VectorSubcoreMesh(core_axis_name="c", subcore_axis_name="s",
                                  num_cores=1)

    @functools.partial(
        pl.kernel,
        out_type=jax.ShapeDtypeStruct((N, D), jnp.float32),
        mesh=mesh,
        compiler_params=pltpu.CompilerParams(needs_layout_passes=False),
        scratch_types=[
            pltpu.VMEM((SCH * K,), jnp.int32),
            pltpu.VMEM((SCH * K,), jnp.int32),
            pltpu.VMEM((K, D), jnp.float32),
            pltpu.VMEM((K, D), jnp.float32),
            pltpu.VMEM((ZROWS, D), jnp.float32),
            pltpu.VMEM((N,), jnp.float32),
            pltpu.VMEM((BLK_LAST,), jnp.float32),
            pltpu.VMEM((BLK_LAST,), jnp.float32),
            pltpu.VMEM((BLK_LAST,), jnp.float32),
            pltpu.VMEM((16, D), jnp.float32),
            pltpu.VMEM_SHARED((N, D), jnp.float32),
            pltpu.VMEM_SHARED((NS * N,), jnp.float32),
            pltpu.SemaphoreType.DMA,
            pltpu.SemaphoreType.DMA,
        ],
    )
    def k(h_hbm, src_hbm, dst_hbm, out_hbm,
          si, di, rows0, rows1, zb, dl, dsum, dtmp, dinv, cbuf,
          agg_sh, deg_st, semg0, semg1):
        rows = (rows0, rows1)
        semg = (semg0, semg1)
        s = lax.axis_index("s")
        zeros16 = jnp.zeros((16,), jnp.float32)
        ones16 = jnp.ones((16,), jnp.float32)

        for i in range(ZROWS):
            for j in range(D // 16):
                zb[i, pl.ds(j * 16, 16)] = zeros16

        def dl_zero(i, _):
            dl[pl.ds(pl.multiple_of(i * 16, 16), 16)] = zeros16
            return 0
        lax.fori_loop(0, N // 16, dl_zero, 0)

        @pl.when(s < WB_TILES)
        def _():
            def zcopy(t, _):
                r = pl.multiple_of(s * WB_ROWS + t * ZROWS, 8)
                pltpu.sync_copy(zb, agg_sh.at[pl.ds(r, ZROWS)])
                return 0
            lax.fori_loop(0, WB_ROWS // ZROWS, zcopy, 0)

        plsc.subcore_barrier()

        base = s * EW

        def superchunk(g, _):
            soff = pl.multiple_of(base + g * (SCH * K), 8)
            pltpu.sync_copy(src_hbm.at[pl.ds(soff, SCH * K)], si)
            pltpu.sync_copy(dst_hbm.at[pl.ds(soff, SCH * K)], di)
            for r in range(2):
                pltpu.async_copy(h_hbm.at[si.at[pl.ds(r * K, K)]],
                                 rows[r], semg[r])
            for b in range(SCH):
                r = b % 2
                pltpu.make_async_copy(h_hbm.at[pl.ds(0, K)], rows[r],
                                      semg[r]).wait()
                pltpu.sync_copy(rows[r], agg_sh.at[di.at[pl.ds(b * K, K)]],
                                add=True)
                for i in range(K // 16):
                    plsc.addupdate_scatter(
                        dl, [di[pl.ds(b * K + i * 16, 16)]], ones16)
                if b + 2 < SCH:
                    pltpu.async_copy(
                        h_hbm.at[si.at[pl.ds((b + 2) * K, K)]],
                        rows[r], semg[r])
            return 0

        lax.fori_loop(0, NSUP, superchunk, 0)

        pltpu.sync_copy(dl, deg_st.at[pl.ds(s * N, N)])
        plsc.subcore_barrier()

        def epilogue(r0, blk):
            pltpu.sync_copy(deg_st.at[pl.ds(r0, blk)], dsum.at[pl.ds(0, blk)])

            def acc_t(t, _):
                o = pl.multiple_of(t * N + r0, 8)
                pltpu.sync_copy(deg_st.at[pl.ds(o, blk)], dtmp.at[pl.ds(0, blk)])
                for i in range(blk // 16):
                    dsum[pl.ds(i * 16, 16)] = (dsum[pl.ds(i * 16, 16)]
                                               + dtmp[pl.ds(i * 16, 16)])
                return 0
            lax.fori_loop(1, NS, acc_t, 0)

            for i in range(blk // 16):
                dinv[pl.ds(i * 16, 16)] = 1.0 / jnp.maximum(
                    dsum[pl.ds(i * 16, 16)], 1.0)

            def rowblk(g, _):
                rb = pl.multiple_of(r0 + g * 16, 8)
                pltpu.sync_copy(agg_sh.at[pl.ds(rb, 16)], cbuf)
                for l in range(16):
                    bv = plsc.load_gather(
                        dinv, [jnp.full((16,), g * 16 + l, jnp.int32)])
                    for j in range(D // 16):
                        cbuf[l, pl.ds(j * 16, 16)] = (
                            cbuf[l, pl.ds(j * 16, 16)] * bv)
                pltpu.sync_copy(cbuf, out_hbm.at[pl.ds(rb, 16)])
                return 0
            lax.fori_loop(0, blk // 16, rowblk, 0)

        @pl.when(s < 15)
        def _():
            epilogue(pl.multiple_of(s * BLK, 8), BLK)

        @pl.when(s == 15)
        def _():
            epilogue(15 * BLK, BLK_LAST)

    return k(h, src, dst)


def _tc_body(h_ref, c_ref, w1_ref, w2_ref, b_ref, g_ref, be_ref, out_ref):
    h = h_ref[...]
    z = (jnp.dot(h, w1_ref[...], preferred_element_type=jnp.float32)
         + jnp.dot(c_ref[...], w2_ref[...], preferred_element_type=jnp.float32)
         + b_ref[...])
    n2 = jnp.sum(z * z, axis=1, keepdims=True)
    z = z * lax.rsqrt(jnp.maximum(n2, 1e-24))
    r = jnp.maximum(z, 0.0)
    mean = jnp.mean(r, axis=0, keepdims=True)
    var = jnp.mean((r - mean) * (r - mean), axis=0, keepdims=True)
    out_ref[...] = (h + (r - mean) * lax.rsqrt(var + 1e-5) * g_ref[...]
                    + be_ref[...])


def kernel(h, edge_index, W, b, gamma, beta):
    src = edge_index[0]
    dst = edge_index[1]
    cfeat = _sc_segment_mean(h, src, dst)
    w1 = W[:, :D].T
    w2 = W[:, D:].T
    out = pl.pallas_call(
        _tc_body,
        out_shape=jax.ShapeDtypeStruct((N, D), jnp.float32),
    )(h, cfeat, w1, w2,
      b.reshape(1, D), gamma.reshape(1, D), beta.reshape(1, D))
    return out


# speedup vs baseline: 9.5524x; 1.2397x over previous
import functools

import jax
import jax.numpy as jnp
from jax import lax
from jax.experimental import pallas as pl
from jax.experimental.pallas import tpu as pltpu
from jax.experimental.pallas import tpu_sc as plsc

N = 10000
E = 320000
D = 128

NC = 2
NS = 16
NW = NC * NS
EW = E // NW
K = 80
NCHUNK = EW // K
SCH = 25
NSUP = NCHUNK // SCH
WB_TILES = 10
WB_ROWS = N // WB_TILES
ZROWS = 8
CB = 400
CW = N // CB


def _sc_partials(h, src, dst):
    mesh = plsc.VectorSubcoreMesh(core_axis_name="c", subcore_axis_name="s")

    @functools.partial(
        pl.kernel,
        out_type=(
            jax.ShapeDtypeStruct((NC, N, D), jnp.float32),
            jax.ShapeDtypeStruct((NW * N,), jnp.float32),
        ),
        mesh=mesh,
        compiler_params=pltpu.CompilerParams(needs_layout_passes=False),
        scratch_types=[
            pltpu.VMEM((SCH * K,), jnp.int32),
            pltpu.VMEM((SCH * K,), jnp.int32),
            pltpu.VMEM((K, D), jnp.float32),
            pltpu.VMEM((K, D), jnp.float32),
            pltpu.VMEM((ZROWS, D), jnp.float32),
            pltpu.VMEM((N,), jnp.float32),
            pltpu.VMEM_SHARED((N, D), jnp.float32),
            pltpu.SemaphoreType.DMA,
            pltpu.SemaphoreType.DMA,
        ],
    )
    def k(h_hbm, src_hbm, dst_hbm, agg_out, deg_out,
          si, di, rows0, rows1, zb, dl, agg_sh, semg0, semg1):
        rows = (rows0, rows1)
        semg = (semg0, semg1)
        c = lax.axis_index("c")
        s = lax.axis_index("s")
        w = c * NS + s
        zeros16 = jnp.zeros((16,), jnp.float32)
        ones16 = jnp.ones((16,), jnp.float32)

        for i in range(ZROWS):
            for j in range(D // 16):
                zb[i, pl.ds(j * 16, 16)] = zeros16

        def dl_zero(i, _):
            dl[pl.ds(pl.multiple_of(i * 16, 16), 16)] = zeros16
            return 0
        lax.fori_loop(0, N // 16, dl_zero, 0)

        @pl.when(s < WB_TILES)
        def _():
            def zcopy(t, _):
                r = pl.multiple_of(s * WB_ROWS + t * ZROWS, 8)
                pltpu.sync_copy(zb, agg_sh.at[pl.ds(r, ZROWS)])
                return 0
            lax.fori_loop(0, WB_ROWS // ZROWS, zcopy, 0)

        plsc.subcore_barrier()

        base = w * EW

        def superchunk(g, _):
            soff = pl.multiple_of(base + g * (SCH * K), 8)
            pltpu.sync_copy(src_hbm.at[pl.ds(soff, SCH * K)], si)
            pltpu.sync_copy(dst_hbm.at[pl.ds(soff, SCH * K)], di)
            for r in range(2):
                pltpu.async_copy(h_hbm.at[si.at[pl.ds(r * K, K)]],
                                 rows[r], semg[r])
            for b in range(SCH):
                r = b % 2
                pltpu.make_async_copy(h_hbm.at[pl.ds(0, K)], rows[r],
                                      semg[r]).wait()
                pltpu.sync_copy(rows[r], agg_sh.at[di.at[pl.ds(b * K, K)]],
                                add=True)
                for i in range(K // 16):
                    plsc.addupdate_scatter(
                        dl, [di[pl.ds(b * K + i * 16, 16)]], ones16)
                if b + 2 < SCH:
                    pltpu.async_copy(
                        h_hbm.at[si.at[pl.ds((b + 2) * K, K)]],
                        rows[r], semg[r])
            return 0

        lax.fori_loop(0, NSUP, superchunk, 0)

        pltpu.sync_copy(dl, deg_out.at[pl.ds(w * N, N)])
        plsc.subcore_barrier()

        @pl.when(s < WB_TILES)
        def _():
            r0 = s * WB_ROWS
            pltpu.sync_copy(agg_sh.at[pl.ds(r0, WB_ROWS)],
                            agg_out.at[c, pl.ds(r0, WB_ROWS)])

    return k(h, src, dst)


def _sc_combine(agg_part, deg_part):
    mesh = plsc.VectorSubcoreMesh(core_axis_name="c", subcore_axis_name="s")

    @functools.partial(
        pl.kernel,
        out_type=jax.ShapeDtypeStruct((N, D), jnp.float32),
        mesh=mesh,
        compiler_params=pltpu.CompilerParams(needs_layout_passes=False),
        scratch_types=[
            pltpu.VMEM((CB,), jnp.float32),
            pltpu.VMEM((CB,), jnp.float32),
            pltpu.VMEM((CB,), jnp.float32),
            pltpu.VMEM((16, D), jnp.float32),
            pltpu.VMEM((16, D), jnp.float32),
        ],
    )
    def k(agg_hbm, deg_hbm, out_hbm, dsum, dtmp, dinv, cb0, cb1):
        c = lax.axis_index("c")
        s = lax.axis_index("s")
        w = c * NS + s

        @pl.when(w < CW)
        def _():
            r0 = pl.multiple_of(w * CB, 8)
            pltpu.sync_copy(deg_hbm.at[pl.ds(r0, CB)], dsum)

            def acc_t(t, _):
                o = pl.multiple_of(t * N + r0, 8)
                pltpu.sync_copy(deg_hbm.at[pl.ds(o, CB)], dtmp)
                for i in range(CB // 16):
                    dsum[pl.ds(i * 16, 16)] = (dsum[pl.ds(i * 16, 16)]
                                               + dtmp[pl.ds(i * 16, 16)])
                return 0
            lax.fori_loop(1, NW, acc_t, 0)

            for i in range(CB // 16):
                dinv[pl.ds(i * 16, 16)] = 1.0 / jnp.maximum(
                    dsum[pl.ds(i * 16, 16)], 1.0)

            def rowblk(g, _):
                rb = pl.multiple_of(r0 + g * 16, 8)
                pltpu.sync_copy(agg_hbm.at[0, pl.ds(rb, 16)], cb0)
                pltpu.sync_copy(agg_hbm.at[1, pl.ds(rb, 16)], cb1)
                for l in range(16):
                    bv = plsc.load_gather(
                        dinv, [jnp.full((16,), g * 16 + l, jnp.int32)])
                    for j in range(D // 16):
                        cb0[l, pl.ds(j * 16, 16)] = (
                            cb0[l, pl.ds(j * 16, 16)]
                            + cb1[l, pl.ds(j * 16, 16)]) * bv
                pltpu.sync_copy(cb0, out_hbm.at[pl.ds(rb, 16)])
                return 0
            lax.fori_loop(0, CB // 16, rowblk, 0)

    return k(agg_part, deg_part)


def _tc_body(h_ref, c_ref, w1_ref, w2_ref, b_ref, g_ref, be_ref, out_ref):
    h = h_ref[...]
    z = (jnp.dot(h, w1_ref[...], preferred_element_type=jnp.float32)
         + jnp.dot(c_ref[...], w2_ref[...], preferred_element_type=jnp.float32)
         + b_ref[...])
    n2 = jnp.sum(z * z, axis=1, keepdims=True)
    z = z * lax.rsqrt(jnp.maximum(n2, 1e-24))
    r = jnp.maximum(z, 0.0)
    mean = jnp.mean(r, axis=0, keepdims=True)
    var = jnp.mean((r - mean) * (r - mean), axis=0, keepdims=True)
    out_ref[...] = (h + (r - mean) * lax.rsqrt(var + 1e-5) * g_ref[...]
                    + be_ref[...])


def kernel(h, edge_index, W, b, gamma, beta):
    src = edge_index[0]
    dst = edge_index[1]
    agg_part, deg_part = _sc_partials(h, src, dst)
    cfeat = _sc_combine(agg_part, deg_part)
    w1 = W[:, :D].T
    w2 = W[:, D:].T
    out = pl.pallas_call(
        _tc_body,
        out_shape=jax.ShapeDtypeStruct((N, D), jnp.float32),
    )(h, cfeat, w1, w2,
      b.reshape(1, D), gamma.reshape(1, D), beta.reshape(1, D))
    return out


# speedup vs baseline: 10.4492x; 1.0939x over previous
import functools

import jax
import jax.numpy as jnp
from jax import lax
from jax.experimental import pallas as pl
from jax.experimental.pallas import tpu as pltpu
from jax.experimental.pallas import tpu_sc as plsc

N = 10000
E = 320000
D = 128

NC = 2
NS = 16
NW = NC * NS
EW = E // NW
K = 80
NCHUNK = EW // K
SCH = 25
NSUP = NCHUNK // SCH
WB_TILES = 10
WB_ROWS = N // WB_TILES
ZROWS = 8
CB = 400
CW = N // CB


def _sc_partials(h, src, dst):
    mesh = plsc.VectorSubcoreMesh(core_axis_name="c", subcore_axis_name="s")

    @functools.partial(
        pl.kernel,
        out_type=(
            jax.ShapeDtypeStruct((NC, N, D), jnp.float32),
            jax.ShapeDtypeStruct((NW * N,), jnp.float32),
        ),
        mesh=mesh,
        compiler_params=pltpu.CompilerParams(needs_layout_passes=False),
        scratch_types=[
            pltpu.VMEM((SCH * K,), jnp.int32),
            pltpu.VMEM((SCH * K,), jnp.int32),
            pltpu.VMEM((K, D), jnp.float32),
            pltpu.VMEM((K, D), jnp.float32),
            pltpu.VMEM((K, D), jnp.float32),
            pltpu.VMEM((ZROWS, D), jnp.float32),
            pltpu.VMEM((N,), jnp.float32),
            pltpu.VMEM_SHARED((N, D), jnp.float32),
            pltpu.SemaphoreType.DMA,
            pltpu.SemaphoreType.DMA,
            pltpu.SemaphoreType.DMA,
        ],
    )
    def k(h_hbm, src_hbm, dst_hbm, agg_out, deg_out,
          si, di, rows0, rows1, rows2, zb, dl, agg_sh, semg0, semg1, semg2):
        rows = (rows0, rows1, rows2)
        semg = (semg0, semg1, semg2)
        c = lax.axis_index("c")
        s = lax.axis_index("s")
        w = c * NS + s
        zeros16 = jnp.zeros((16,), jnp.float32)
        ones16 = jnp.ones((16,), jnp.float32)

        for i in range(ZROWS):
            for j in range(D // 16):
                zb[i, pl.ds(j * 16, 16)] = zeros16

        def dl_zero(i, _):
            dl[pl.ds(pl.multiple_of(i * 16, 16), 16)] = zeros16
            return 0
        lax.fori_loop(0, N // 16, dl_zero, 0)

        @pl.when(s < WB_TILES)
        def _():
            def zcopy(t, _):
                r = pl.multiple_of(s * WB_ROWS + t * ZROWS, 8)
                pltpu.sync_copy(zb, agg_sh.at[pl.ds(r, ZROWS)])
                return 0
            lax.fori_loop(0, WB_ROWS // ZROWS, zcopy, 0)

        plsc.subcore_barrier()

        base = w * EW

        def superchunk(g, _):
            soff = pl.multiple_of(base + g * (SCH * K), 8)
            pltpu.sync_copy(src_hbm.at[pl.ds(soff, SCH * K)], si)
            pltpu.sync_copy(dst_hbm.at[pl.ds(soff, SCH * K)], di)
            for r in range(3):
                pltpu.async_copy(h_hbm.at[si.at[pl.ds(r * K, K)]],
                                 rows[r], semg[r])
            for b in range(SCH):
                r = b % 3
                pltpu.make_async_copy(h_hbm.at[pl.ds(0, K)], rows[r],
                                      semg[r]).wait()
                pltpu.sync_copy(rows[r], agg_sh.at[di.at[pl.ds(b * K, K)]],
                                add=True)
                for i in range(K // 16):
                    plsc.addupdate_scatter(
                        dl, [di[pl.ds(b * K + i * 16, 16)]], ones16)
                if b + 3 < SCH:
                    pltpu.async_copy(
                        h_hbm.at[si.at[pl.ds((b + 3) * K, K)]],
                        rows[r], semg[r])
            return 0

        lax.fori_loop(0, NSUP, superchunk, 0)

        pltpu.sync_copy(dl, deg_out.at[pl.ds(w * N, N)])
        plsc.subcore_barrier()

        @pl.when(s < WB_TILES)
        def _():
            r0 = s * WB_ROWS
            pltpu.sync_copy(agg_sh.at[pl.ds(r0, WB_ROWS)],
                            agg_out.at[c, pl.ds(r0, WB_ROWS)])

    return k(h, src, dst)


def _sc_combine(agg_part, deg_part):
    mesh = plsc.VectorSubcoreMesh(core_axis_name="c", subcore_axis_name="s")

    @functools.partial(
        pl.kernel,
        out_type=jax.ShapeDtypeStruct((N, D), jnp.float32),
        mesh=mesh,
        compiler_params=pltpu.CompilerParams(needs_layout_passes=False),
        scratch_types=[
            pltpu.VMEM((CB,), jnp.float32),
            pltpu.VMEM((CB,), jnp.float32),
            pltpu.VMEM((CB,), jnp.float32),
            pltpu.VMEM((16, D), jnp.float32),
            pltpu.VMEM((16, D), jnp.float32),
        ],
    )
    def k(agg_hbm, deg_hbm, out_hbm, dsum, dtmp, dinv, cb0, cb1):
        c = lax.axis_index("c")
        s = lax.axis_index("s")
        w = c * NS + s

        @pl.when(w < CW)
        def _():
            r0 = pl.multiple_of(w * CB, 8)
            pltpu.sync_copy(deg_hbm.at[pl.ds(r0, CB)], dsum)

            def acc_t(t, _):
                o = pl.multiple_of(t * N + r0, 8)
                pltpu.sync_copy(deg_hbm.at[pl.ds(o, CB)], dtmp)
                for i in range(CB // 16):
                    dsum[pl.ds(i * 16, 16)] = (dsum[pl.ds(i * 16, 16)]
                                               + dtmp[pl.ds(i * 16, 16)])
                return 0
            lax.fori_loop(1, NW, acc_t, 0)

            for i in range(CB // 16):
                dinv[pl.ds(i * 16, 16)] = 1.0 / jnp.maximum(
                    dsum[pl.ds(i * 16, 16)], 1.0)

            def rowblk(g, _):
                rb = pl.multiple_of(r0 + g * 16, 8)
                pltpu.sync_copy(agg_hbm.at[0, pl.ds(rb, 16)], cb0)
                pltpu.sync_copy(agg_hbm.at[1, pl.ds(rb, 16)], cb1)
                for l in range(16):
                    bv = plsc.load_gather(
                        dinv, [jnp.full((16,), g * 16 + l, jnp.int32)])
                    for j in range(D // 16):
                        cb0[l, pl.ds(j * 16, 16)] = (
                            cb0[l, pl.ds(j * 16, 16)]
                            + cb1[l, pl.ds(j * 16, 16)]) * bv
                pltpu.sync_copy(cb0, out_hbm.at[pl.ds(rb, 16)])
                return 0
            lax.fori_loop(0, CB // 16, rowblk, 0)

    return k(agg_part, deg_part)


def _tc_body(h_ref, c_ref, w1_ref, w2_ref, b_ref, g_ref, be_ref, out_ref):
    h = h_ref[...]
    z = (jnp.dot(h, w1_ref[...], preferred_element_type=jnp.float32)
         + jnp.dot(c_ref[...], w2_ref[...], preferred_element_type=jnp.float32)
         + b_ref[...])
    n2 = jnp.sum(z * z, axis=1, keepdims=True)
    z = z * lax.rsqrt(jnp.maximum(n2, 1e-24))
    r = jnp.maximum(z, 0.0)
    mean = jnp.mean(r, axis=0, keepdims=True)
    var = jnp.mean((r - mean) * (r - mean), axis=0, keepdims=True)
    out_ref[...] = (h + (r - mean) * lax.rsqrt(var + 1e-5) * g_ref[...]
                    + be_ref[...])


def kernel(h, edge_index, W, b, gamma, beta):
    src = edge_index[0]
    dst = edge_index[1]
    agg_part, deg_part = _sc_partials(h, src, dst)
    cfeat = _sc_combine(agg_part, deg_part)
    w1 = W[:, :D].T
    w2 = W[:, D:].T
    out = pl.pallas_call(
        _tc_body,
        out_shape=jax.ShapeDtypeStruct((N, D), jnp.float32),
    )(h, cfeat, w1, w2,
      b.reshape(1, D), gamma.reshape(1, D), beta.reshape(1, D))
    return out


# speedup vs baseline: 11.3642x; 1.0876x over previous
import functools

import jax
import jax.numpy as jnp
from jax import lax
from jax.experimental import pallas as pl
from jax.experimental.pallas import tpu as pltpu
from jax.experimental.pallas import tpu_sc as plsc

N = 10000
E = 320000
D = 128

NC = 2
NS = 16
NW = NC * NS
EW = E // NW
K = 80
NCHUNK = EW // K
SCH = 25
NSUP = NCHUNK // SCH
WB_TILES = 10
WB_ROWS = N // WB_TILES
ZROWS = 8
CB = 400
CW = N // CB


def _sc_partials(h, src, dst):
    mesh = plsc.VectorSubcoreMesh(core_axis_name="c", subcore_axis_name="s")

    @functools.partial(
        pl.kernel,
        out_type=(
            jax.ShapeDtypeStruct((NC, N, D), jnp.float32),
            jax.ShapeDtypeStruct((NW * N,), jnp.float32),
        ),
        mesh=mesh,
        compiler_params=pltpu.CompilerParams(needs_layout_passes=False),
        scratch_types=[
            pltpu.VMEM((SCH * K,), jnp.int32),
            pltpu.VMEM((SCH * K,), jnp.int32),
            pltpu.VMEM((K, D), jnp.float32),
            pltpu.VMEM((K, D), jnp.float32),
            pltpu.VMEM((K, D), jnp.float32),
            pltpu.VMEM((ZROWS, D), jnp.float32),
            pltpu.VMEM((N,), jnp.float32),
            pltpu.VMEM_SHARED((N, D), jnp.float32),
            pltpu.SemaphoreType.DMA,
            pltpu.SemaphoreType.DMA,
            pltpu.SemaphoreType.DMA,
        ],
    )
    def k(h_hbm, src_hbm, dst_hbm, agg_out, deg_out,
          si, di, rows0, rows1, rows2, zb, dl, agg_sh, semg0, semg1, semg2):
        rows = (rows0, rows1, rows2)
        semg = (semg0, semg1, semg2)
        c = lax.axis_index("c")
        s = lax.axis_index("s")
        w = c * NS + s
        zeros16 = jnp.zeros((16,), jnp.float32)
        ones16 = jnp.ones((16,), jnp.float32)

        for i in range(ZROWS):
            for j in range(D // 16):
                zb[i, pl.ds(j * 16, 16)] = zeros16

        def dl_zero(i, _):
            dl[pl.ds(pl.multiple_of(i * 16, 16), 16)] = zeros16
            return 0
        lax.fori_loop(0, N // 16, dl_zero, 0)

        @pl.when(s < WB_TILES)
        def _():
            def zcopy(t, _):
                r = pl.multiple_of(s * WB_ROWS + t * ZROWS, 8)
                pltpu.sync_copy(zb, agg_sh.at[pl.ds(r, ZROWS)])
                return 0
            lax.fori_loop(0, WB_ROWS // ZROWS, zcopy, 0)

        plsc.subcore_barrier()

        base = w * EW

        def superchunk(g, _):
            soff = pl.multiple_of(base + g * (SCH * K), 8)
            pltpu.sync_copy(src_hbm.at[pl.ds(soff, SCH * K)], si)
            pltpu.sync_copy(dst_hbm.at[pl.ds(soff, SCH * K)], di)
            for r in range(3):
                pltpu.async_copy(h_hbm.at[si.at[pl.ds(r * K, K)]],
                                 rows[r], semg[r])
            for b in range(SCH):
                r = b % 3
                pltpu.make_async_copy(h_hbm.at[pl.ds(0, K)], rows[r],
                                      semg[r]).wait()
                pltpu.sync_copy(rows[r], agg_sh.at[di.at[pl.ds(b * K, K)]],
                                add=True)
                for i in range(K // 16):
                    plsc.addupdate_scatter(
                        dl, [di[pl.ds(b * K + i * 16, 16)]], ones16)
                if b + 3 < SCH:
                    pltpu.async_copy(
                        h_hbm.at[si.at[pl.ds((b + 3) * K, K)]],
                        rows[r], semg[r])
            return 0

        lax.fori_loop(0, NSUP, superchunk, 0)

        pltpu.sync_copy(dl, deg_out.at[pl.ds(w * N, N)])
        plsc.subcore_barrier()

        @pl.when(s < WB_TILES)
        def _():
            r0 = s * WB_ROWS
            pltpu.sync_copy(agg_sh.at[pl.ds(r0, WB_ROWS)],
                            agg_out.at[c, pl.ds(r0, WB_ROWS)])

    return k(h, src, dst)


def _sc_combine(agg_part, deg_part):
    mesh = plsc.VectorSubcoreMesh(core_axis_name="c", subcore_axis_name="s")

    @functools.partial(
        pl.kernel,
        out_type=jax.ShapeDtypeStruct((N, D), jnp.float32),
        mesh=mesh,
        compiler_params=pltpu.CompilerParams(needs_layout_passes=False),
        scratch_types=[
            pltpu.VMEM((CB,), jnp.float32),
            pltpu.VMEM((CB,), jnp.float32),
            pltpu.VMEM((CB,), jnp.float32),
            pltpu.VMEM((16, D), jnp.float32),
            pltpu.VMEM((16, D), jnp.float32),
            pltpu.VMEM((16, D), jnp.float32),
            pltpu.VMEM((16, D), jnp.float32),
            pltpu.SemaphoreType.DMA,
            pltpu.SemaphoreType.DMA,
            pltpu.SemaphoreType.DMA,
            pltpu.SemaphoreType.DMA,
        ],
    )
    def k(agg_hbm, deg_hbm, out_hbm, dsum, dtmp, dinv,
          cb0a, cb1a, cb0b, cb1b, seml0, seml1, sems0, sems1):
        c = lax.axis_index("c")
        s = lax.axis_index("s")
        w = c * NS + s
        cb0 = (cb0a, cb0b)
        cb1 = (cb1a, cb1b)
        seml = (seml0, seml1)
        sems = (sems0, sems1)
        NB = CB // 16

        @pl.when(w < CW)
        def _():
            r0 = pl.multiple_of(w * CB, 8)
            pltpu.sync_copy(deg_hbm.at[pl.ds(r0, CB)], dsum)

            def acc_t(t, _):
                o = pl.multiple_of(t * N + r0, 8)
                pltpu.sync_copy(deg_hbm.at[pl.ds(o, CB)], dtmp)
                for i in range(CB // 16):
                    dsum[pl.ds(i * 16, 16)] = (dsum[pl.ds(i * 16, 16)]
                                               + dtmp[pl.ds(i * 16, 16)])
                return 0
            lax.fori_loop(1, NW, acc_t, 0)

            for i in range(CB // 16):
                dinv[pl.ds(i * 16, 16)] = 1.0 / jnp.maximum(
                    dsum[pl.ds(i * 16, 16)], 1.0)

            def fire_loads(g, r):
                rb = pl.multiple_of(r0 + g * 16, 8)
                pltpu.async_copy(agg_hbm.at[0, pl.ds(rb, 16)], cb0[r], seml[r])
                pltpu.async_copy(agg_hbm.at[1, pl.ds(rb, 16)], cb1[r], seml[r])

            def wait_loads(r):
                pltpu.make_async_copy(agg_hbm.at[0, pl.ds(0, 16)], cb0[r],
                                      seml[r]).wait()
                pltpu.make_async_copy(agg_hbm.at[0, pl.ds(0, 16)], cb1[r],
                                      seml[r]).wait()

            def wait_store(r):
                pltpu.make_async_copy(cb0[r], out_hbm.at[pl.ds(0, 16)],
                                      sems[r]).wait()

            def compute_store(g, r):
                for l in range(16):
                    bv = plsc.load_gather(
                        dinv, [jnp.full((16,), g * 16 + l, jnp.int32)])
                    for j in range(D // 16):
                        cb0[r][l, pl.ds(j * 16, 16)] = (
                            cb0[r][l, pl.ds(j * 16, 16)]
                            + cb1[r][l, pl.ds(j * 16, 16)]) * bv
                rb = pl.multiple_of(r0 + g * 16, 8)
                pltpu.async_copy(cb0[r], out_hbm.at[pl.ds(rb, 16)], sems[r])

            fire_loads(0, 0)

            def pair(p, _):
                wait_loads(0)

                @pl.when(p > 0)
                def _():
                    wait_store(1)
                fire_loads(2 * p + 1, 1)
                compute_store(2 * p, 0)
                wait_loads(1)
                wait_store(0)
                fire_loads(2 * p + 2, 0)
                compute_store(2 * p + 1, 1)
                return 0

            lax.fori_loop(0, (NB - 1) // 2, pair, 0)

            wait_loads(0)
            compute_store(NB - 1, 0)
            wait_store(0)
            wait_store(1)

    return k(agg_part, deg_part)


def _tc_body(h_ref, c_ref, w1_ref, w2_ref, b_ref, g_ref, be_ref, out_ref):
    h = h_ref[...]
    z = (jnp.dot(h, w1_ref[...], preferred_element_type=jnp.float32)
         + jnp.dot(c_ref[...], w2_ref[...], preferred_element_type=jnp.float32)
         + b_ref[...])
    n2 = jnp.sum(z * z, axis=1, keepdims=True)
    z = z * lax.rsqrt(jnp.maximum(n2, 1e-24))
    r = jnp.maximum(z, 0.0)
    mean = jnp.mean(r, axis=0, keepdims=True)
    var = jnp.mean((r - mean) * (r - mean), axis=0, keepdims=True)
    out_ref[...] = (h + (r - mean) * lax.rsqrt(var + 1e-5) * g_ref[...]
                    + be_ref[...])


def kernel(h, edge_index, W, b, gamma, beta):
    src = edge_index[0]
    dst = edge_index[1]
    agg_part, deg_part = _sc_partials(h, src, dst)
    cfeat = _sc_combine(agg_part, deg_part)
    w1 = W[:, :D].T
    w2 = W[:, D:].T
    out = pl.pallas_call(
        _tc_body,
        out_shape=jax.ShapeDtypeStruct((N, D), jnp.float32),
    )(h, cfeat, w1, w2,
      b.reshape(1, D), gamma.reshape(1, D), beta.reshape(1, D))
    return out
